# baseline (speedup 1.0000x reference)
import functools

import jax
import jax.numpy as jnp
from jax import lax
from jax.experimental import pallas as pl
from jax.experimental.pallas import tpu as pltpu
from jax.experimental.pallas import tpu_sc as plsc

_B = 16384
_D = 64
_L = 16
_HALF = 8192
_NC = 2
_NS = 16
_NW = _NC * _NS
_RPW = _B // _NW
_CHUNK = 128
_NCHUNK = _RPW // _CHUNK
_GPC = _CHUNK // _L

_TBLK = 1024


def _transpose_body(x_ref, o_ref):
    xt = jnp.transpose(x_ref[...], (1, 0))
    y = xt.reshape(_TBLK // 2, 2, _D)
    o_ref[:, 0:_D] = y[:, 0, :]
    o_ref[:, _D:2 * _D] = y[:, 1, :]


def _pack_pairs(table_t):
    n = table_t.shape[1]
    grid = (n + _TBLK - 1) // _TBLK
    return pl.pallas_call(
        _transpose_body,
        grid=(grid,),
        in_specs=[pl.BlockSpec((_D, _TBLK), lambda i: (0, i))],
        out_specs=pl.BlockSpec((_TBLK // 2, 2 * _D), lambda i: (i, 0)),
        out_shape=jax.ShapeDtypeStruct((n // 2, 2 * _D), jnp.float32),
    )(table_t)


def _sc_body(hq, tq, rq, hp, tp, rp, ent_hbm, rel_hbm, out_hbm,
             qh_v, qt_v, qr_v, ph_v, pt_v, pr_v,
             rows_h, rows_t, rows_r, p_v, sem):
    wid = lax.axis_index("s") * _NC + lax.axis_index("c")

    pltpu.sync_copy(hq.at[wid], qh_v)
    pltpu.sync_copy(tq.at[wid], qt_v)
    pltpu.sync_copy(rq.at[wid], qr_v)
    pltpu.sync_copy(hp.at[wid], ph_v)
    pltpu.sync_copy(tp.at[wid], pt_v)
    pltpu.sync_copy(rp.at[wid], pr_v)

    for c in range(_NCHUNK):
        cps = [
            pltpu.async_copy(ent_hbm.at[qh_v.at[c]], rows_h, sem),
            pltpu.async_copy(ent_hbm.at[qt_v.at[c]], rows_t, sem),
            pltpu.async_copy(rel_hbm.at[qr_v.at[c]], rows_r, sem),
        ]
        for cp in cps:
            cp.wait()

        def group(g, _):
            gsl = pl.ds(g * _L, _L)
            pvh = ph_v[c, gsl]
            pvt = pt_v[c, gsl]
            pvr = pr_v[c, gsl]
            for rr in range(_L):
                row = g * _L + rr
                oh = pvh[rr]
                ot = pvt[rr]
                orr = pvr[rr]
                s = None
                for k in range(_D // _L):
                    dh = rows_h[row, pl.ds(oh + k * _L, _L)]
                    dr = rows_r[row, pl.ds(orr + k * _L, _L)]
                    dt = rows_t[row, pl.ds(ot + k * _L, _L)]
                    d = dh + dr - dt
                    sq = d * d
                    s = sq if s is None else s + sq
                p_v[pl.ds((c * _CHUNK + row) * _L, _L)] = s
            return 0

        lax.fori_loop(0, _GPC, group, 0)

    pltpu.sync_copy(p_v, out_hbm.at[pl.ds(wid * _RPW * _L, _RPW * _L)])


@functools.partial(
    pl.kernel,
    out_type=jax.ShapeDtypeStruct((_B * _L,), jnp.float32),
    mesh=plsc.VectorSubcoreMesh(core_axis_name="c", subcore_axis_name="s"),
    scratch_types=[
        pltpu.VMEM((_NCHUNK, _CHUNK), jnp.int32),
        pltpu.VMEM((_NCHUNK, _CHUNK), jnp.int32),
        pltpu.VMEM((_NCHUNK, _CHUNK), jnp.int32),
        pltpu.VMEM((_NCHUNK, _CHUNK), jnp.int32),
        pltpu.VMEM((_NCHUNK, _CHUNK), jnp.int32),
        pltpu.VMEM((_NCHUNK, _CHUNK), jnp.int32),
        pltpu.VMEM((_CHUNK, 2 * _D), jnp.float32),
        pltpu.VMEM((_CHUNK, 2 * _D), jnp.float32),
        pltpu.VMEM((_CHUNK, 2 * _D), jnp.float32),
        pltpu.VMEM((_RPW * _L,), jnp.float32),
        pltpu.SemaphoreType.DMA,
    ],
)
def _transe_partials(hq, tq, rq, hp, tp, rp, ent_hbm, rel_hbm, out_hbm,
                     qh_v, qt_v, qr_v, ph_v, pt_v, pr_v,
                     rows_h, rows_t, rows_r, p_v, sem):
    _sc_body(hq, tq, rq, hp, tp, rp, ent_hbm, rel_hbm, out_hbm,
             qh_v, qt_v, qr_v, ph_v, pt_v, pr_v,
             rows_h, rows_t, rows_r, p_v, sem)


def _fold_body(p_ref, o_ref):
    o_ref[...] = jnp.sqrt(jnp.sum(p_ref[...], axis=-1))


_fold_sqrt = pl.pallas_call(
    _fold_body,
    out_shape=jax.ShapeDtypeStruct((_B,), jnp.float32),
)


def kernel(h, r, t, batch_size, ent_emb, rel_emb):
    del batch_size
    h = h.astype(jnp.int32)
    t = t.astype(jnp.int32)
    r = r.astype(jnp.int32)
    shape3 = (_NW, _NCHUNK, _CHUNK)
    hq = (h >> 1).reshape(shape3)
    tq = (t >> 1).reshape(shape3)
    rq = (r >> 1).reshape(shape3)
    hp = ((h & 1) * _D).reshape(shape3)
    tp = ((t & 1) * _D).reshape(shape3)
    rp = ((r & 1) * _D).reshape(shape3)
    ent2 = _pack_pairs(ent_emb.T)
    rel2 = _pack_pairs(rel_emb.T)
    partials = _transe_partials(hq, tq, rq, hp, tp, rp, ent2, rel2)
    score = _fold_sqrt(partials.reshape(_B, _L))
    return score[:_HALF], score[_HALF:]


# speedup vs baseline: 1.7438x; 1.7438x over previous
import functools

import jax
import jax.numpy as jnp
from jax import lax
from jax.experimental import pallas as pl
from jax.experimental.pallas import tpu as pltpu
from jax.experimental.pallas import tpu_sc as plsc

_B = 16384
_D = 64
_L = 16
_HALF = 8192
_NC = 2
_NS = 16
_NW = _NC * _NS
_RPW = _B // _NW
_CHUNK = 128
_NCHUNK = _RPW // _CHUNK
_GPC = _CHUNK // _L

def _transpose_body(xa_ref, xb_ref, o_ref):
    x = jnp.concatenate([xa_ref[...], xb_ref[...]], axis=0)
    ii = lax.broadcasted_iota(jnp.int32, (2 * _D, 2 * _D), 0)
    jj = lax.broadcasted_iota(jnp.int32, (2 * _D, 2 * _D), 1)
    ident = (ii == jj).astype(jnp.float32)
    o_ref[...] = lax.dot_general(x, ident, (((0,), (0,)), ((), ())))


def _pack_split(table_t, half):
    n = table_t.shape[1]
    wblk = min(1024, half)
    nblk = half // wblk
    bmax = (n - 1) // wblk

    return pl.pallas_call(
        _transpose_body,
        grid=(nblk,),
        in_specs=[
            pl.BlockSpec((_D, wblk), lambda i: (0, i)),
            pl.BlockSpec((_D, wblk), lambda i: (0, jnp.minimum(i + nblk, bmax))),
        ],
        out_specs=pl.BlockSpec((wblk, 2 * _D), lambda i: (i, 0)),
        out_shape=jax.ShapeDtypeStruct((half, 2 * _D), jnp.float32),
    )(table_t, table_t)


def _sc_body(hq, tq, rq, hp, tp, rp, ent_hbm, rel_hbm, out_hbm,
             qh_v, qt_v, qr_v, ph_v, pt_v, pr_v,
             rows_h, rows_t, rows_r, p_v, sem):
    wid = lax.axis_index("s") * _NC + lax.axis_index("c")

    pltpu.sync_copy(hq.at[wid], qh_v)
    pltpu.sync_copy(tq.at[wid], qt_v)
    pltpu.sync_copy(rq.at[wid], qr_v)
    pltpu.sync_copy(hp.at[wid], ph_v)
    pltpu.sync_copy(tp.at[wid], pt_v)
    pltpu.sync_copy(rp.at[wid], pr_v)

    for c in range(_NCHUNK):
        cps = [
            pltpu.async_copy(ent_hbm.at[qh_v.at[c]], rows_h, sem),
            pltpu.async_copy(ent_hbm.at[qt_v.at[c]], rows_t, sem),
            pltpu.async_copy(rel_hbm.at[qr_v.at[c]], rows_r, sem),
        ]
        for cp in cps:
            cp.wait()

        def group(g, _):
            gsl = pl.ds(g * _L, _L)
            pvh = ph_v[c, gsl]
            pvt = pt_v[c, gsl]
            pvr = pr_v[c, gsl]
            for rr in range(_L):
                row = g * _L + rr
                oh = pvh[rr]
                ot = pvt[rr]
                orr = pvr[rr]
                s = None
                for k in range(_D // _L):
                    dh = rows_h[row, pl.ds(oh + k * _L, _L)]
                    dr = rows_r[row, pl.ds(orr + k * _L, _L)]
                    dt = rows_t[row, pl.ds(ot + k * _L, _L)]
                    d = dh + dr - dt
                    sq = d * d
                    s = sq if s is None else s + sq
                p_v[pl.ds((c * _CHUNK + row) * _L, _L)] = s
            return 0

        lax.fori_loop(0, _GPC, group, 0)

    pltpu.sync_copy(p_v, out_hbm.at[pl.ds(wid * _RPW * _L, _RPW * _L)])


@functools.partial(
    pl.kernel,
    out_type=jax.ShapeDtypeStruct((_B * _L,), jnp.float32),
    mesh=plsc.VectorSubcoreMesh(core_axis_name="c", subcore_axis_name="s"),
    scratch_types=[
        pltpu.VMEM((_NCHUNK, _CHUNK), jnp.int32),
        pltpu.VMEM((_NCHUNK, _CHUNK), jnp.int32),
        pltpu.VMEM((_NCHUNK, _CHUNK), jnp.int32),
        pltpu.VMEM((_NCHUNK, _CHUNK), jnp.int32),
        pltpu.VMEM((_NCHUNK, _CHUNK), jnp.int32),
        pltpu.VMEM((_NCHUNK, _CHUNK), jnp.int32),
        pltpu.VMEM((_CHUNK, 2 * _D), jnp.float32),
        pltpu.VMEM((_CHUNK, 2 * _D), jnp.float32),
        pltpu.VMEM((_CHUNK, 2 * _D), jnp.float32),
        pltpu.VMEM((_RPW * _L,), jnp.float32),
        pltpu.SemaphoreType.DMA,
    ],
)
def _transe_partials(hq, tq, rq, hp, tp, rp, ent_hbm, rel_hbm, out_hbm,
                     qh_v, qt_v, qr_v, ph_v, pt_v, pr_v,
                     rows_h, rows_t, rows_r, p_v, sem):
    _sc_body(hq, tq, rq, hp, tp, rp, ent_hbm, rel_hbm, out_hbm,
             qh_v, qt_v, qr_v, ph_v, pt_v, pr_v,
             rows_h, rows_t, rows_r, p_v, sem)


def _fold_body(p_ref, o_ref):
    o_ref[...] = jnp.sqrt(jnp.sum(p_ref[...], axis=-1))


_fold_sqrt = pl.pallas_call(
    _fold_body,
    out_shape=jax.ShapeDtypeStruct((_B,), jnp.float32),
)


def kernel(h, r, t, batch_size, ent_emb, rel_emb):
    del batch_size
    h = h.astype(jnp.int32)
    t = t.astype(jnp.int32)
    r = r.astype(jnp.int32)
    shape3 = (_NW, _NCHUNK, _CHUNK)
    ehalf = 1 << 19
    rhalf = 1 << 9
    hq = (h & (ehalf - 1)).reshape(shape3)
    tq = (t & (ehalf - 1)).reshape(shape3)
    rq = (r & (rhalf - 1)).reshape(shape3)
    hp = ((h >> 19) * _D).reshape(shape3)
    tp = ((t >> 19) * _D).reshape(shape3)
    rp = ((r >> 9) * _D).reshape(shape3)
    ent2 = _pack_split(ent_emb.T, ehalf)
    rel2 = _pack_split(rel_emb.T, rhalf)
    partials = _transe_partials(hq, tq, rq, hp, tp, rp, ent2, rel2)
    score = _fold_sqrt(partials.reshape(_B, _L))
    return score[:_HALF], score[_HALF:]


# speedup vs baseline: 3.1665x; 1.8159x over previous
import functools

import jax
import jax.numpy as jnp
from jax import lax
from jax.experimental import pallas as pl
from jax.experimental.pallas import tpu as pltpu
from jax.experimental.pallas import tpu_sc as plsc

_B = 16384
_D = 64
_L = 16
_HALF = 8192
_NC = 2
_NS = 16
_NW = _NC * _NS
_RPW = _B // _NW
_CHUNK = 128
_NCHUNK = _RPW // _CHUNK
_GPC = _CHUNK // _L

def _transpose_body(xa_ref, xb_ref, o_ref):
    x = jnp.concatenate([xa_ref[...], xb_ref[...]], axis=0)
    ii = lax.broadcasted_iota(jnp.int32, (2 * _D, 2 * _D), 0)
    jj = lax.broadcasted_iota(jnp.int32, (2 * _D, 2 * _D), 1)
    ident = (ii == jj).astype(jnp.float32)
    o_ref[...] = lax.dot_general(x, ident, (((0,), (0,)), ((), ())))


def _pack_split(table_t, half):
    n = table_t.shape[1]
    wblk = min(4096, half)
    nblk = half // wblk
    bmax = (n - 1) // wblk

    return pl.pallas_call(
        _transpose_body,
        grid=(nblk,),
        in_specs=[
            pl.BlockSpec((_D, wblk), lambda i: (0, i)),
            pl.BlockSpec((_D, wblk), lambda i: (0, jnp.minimum(i + nblk, bmax))),
        ],
        out_specs=pl.BlockSpec((wblk, 2 * _D), lambda i: (i, 0)),
        out_shape=jax.ShapeDtypeStruct((half, 2 * _D), jnp.float32),
    )(table_t, table_t)


def _sc_body(hq, tq, rq, hp, tp, rp, ent_hbm, rel_hbm, out_hbm,
             qh_v, qt_v, qr_v, ph_v, pt_v, pr_v,
             rows_h, rows_t, rows_r, p_v, sem):
    wid = lax.axis_index("s") * _NC + lax.axis_index("c")

    pltpu.sync_copy(hq.at[wid], qh_v)
    pltpu.sync_copy(tq.at[wid], qt_v)
    pltpu.sync_copy(rq.at[wid], qr_v)
    pltpu.sync_copy(hp.at[wid], ph_v)
    pltpu.sync_copy(tp.at[wid], pt_v)
    pltpu.sync_copy(rp.at[wid], pr_v)

    for c in range(_NCHUNK):
        cps = [
            pltpu.async_copy(ent_hbm.at[qh_v.at[c]], rows_h, sem),
            pltpu.async_copy(ent_hbm.at[qt_v.at[c]], rows_t, sem),
            pltpu.async_copy(rel_hbm.at[qr_v.at[c]], rows_r, sem),
        ]
        for cp in cps:
            cp.wait()

        def group(g, _):
            gsl = pl.ds(g * _L, _L)
            pvh = ph_v[c, gsl]
            pvt = pt_v[c, gsl]
            pvr = pr_v[c, gsl]
            for rr in range(_L):
                row = g * _L + rr
                oh = pvh[rr]
                ot = pvt[rr]
                orr = pvr[rr]
                s = None
                for k in range(_D // _L):
                    dh = rows_h[row, pl.ds(oh + k * _L, _L)]
                    dr = rows_r[row, pl.ds(orr + k * _L, _L)]
                    dt = rows_t[row, pl.ds(ot + k * _L, _L)]
                    d = dh + dr - dt
                    sq = d * d
                    s = sq if s is None else s + sq
                p_v[pl.ds((c * _CHUNK + row) * _L, _L)] = s
            return 0

        lax.fori_loop(0, _GPC, group, 0)

    pltpu.sync_copy(p_v, out_hbm.at[pl.ds(wid * _RPW * _L, _RPW * _L)])


@functools.partial(
    pl.kernel,
    out_type=jax.ShapeDtypeStruct((_B * _L,), jnp.float32),
    mesh=plsc.VectorSubcoreMesh(core_axis_name="c", subcore_axis_name="s"),
    scratch_types=[
        pltpu.VMEM((_NCHUNK, _CHUNK), jnp.int32),
        pltpu.VMEM((_NCHUNK, _CHUNK), jnp.int32),
        pltpu.VMEM((_NCHUNK, _CHUNK), jnp.int32),
        pltpu.VMEM((_NCHUNK, _CHUNK), jnp.int32),
        pltpu.VMEM((_NCHUNK, _CHUNK), jnp.int32),
        pltpu.VMEM((_NCHUNK, _CHUNK), jnp.int32),
        pltpu.VMEM((_CHUNK, 2 * _D), jnp.float32),
        pltpu.VMEM((_CHUNK, 2 * _D), jnp.float32),
        pltpu.VMEM((_CHUNK, 2 * _D), jnp.float32),
        pltpu.VMEM((_RPW * _L,), jnp.float32),
        pltpu.SemaphoreType.DMA,
    ],
)
def _transe_partials(hq, tq, rq, hp, tp, rp, ent_hbm, rel_hbm, out_hbm,
                     qh_v, qt_v, qr_v, ph_v, pt_v, pr_v,
                     rows_h, rows_t, rows_r, p_v, sem):
    _sc_body(hq, tq, rq, hp, tp, rp, ent_hbm, rel_hbm, out_hbm,
             qh_v, qt_v, qr_v, ph_v, pt_v, pr_v,
             rows_h, rows_t, rows_r, p_v, sem)


def _fold_body(p_ref, o_ref):
    o_ref[...] = jnp.sqrt(jnp.sum(p_ref[...], axis=-1))


_fold_sqrt = pl.pallas_call(
    _fold_body,
    out_shape=jax.ShapeDtypeStruct((_B,), jnp.float32),
)


def kernel(h, r, t, batch_size, ent_emb, rel_emb):
    del batch_size
    h = h.astype(jnp.int32)
    t = t.astype(jnp.int32)
    r = r.astype(jnp.int32)
    shape3 = (_NW, _NCHUNK, _CHUNK)
    ehalf = 1 << 19
    rhalf = 1 << 9
    hq = (h & (ehalf - 1)).reshape(shape3)
    tq = (t & (ehalf - 1)).reshape(shape3)
    rq = (r & (rhalf - 1)).reshape(shape3)
    hp = ((h >> 19) * _D).reshape(shape3)
    tp = ((t >> 19) * _D).reshape(shape3)
    rp = ((r >> 9) * _D).reshape(shape3)
    ent2 = _pack_split(ent_emb.T, ehalf)
    rel2 = _pack_split(rel_emb.T, rhalf)
    partials = _transe_partials(hq, tq, rq, hp, tp, rp, ent2, rel2)
    score = _fold_sqrt(partials.reshape(_B, _L))
    return score[:_HALF], score[_HALF:]


# speedup vs baseline: 3.5657x; 1.1261x over previous
import functools

import jax
import jax.numpy as jnp
from jax import lax
from jax.experimental import pallas as pl
from jax.experimental.pallas import tpu as pltpu
from jax.experimental.pallas import tpu_sc as plsc

_B = 16384
_D = 64
_L = 16
_HALF = 8192
_NC = 2
_NS = 16
_NW = _NC * _NS
_RPW = _B // _NW
_CHUNK = 128
_NCHUNK = _RPW // _CHUNK
_GPC = _CHUNK // _L

def _transpose_body(xa_ref, xb_ref, o_ref):
    x = jnp.concatenate([xa_ref[...], xb_ref[...]], axis=0)
    ii = lax.broadcasted_iota(jnp.int32, (2 * _D, 2 * _D), 0)
    jj = lax.broadcasted_iota(jnp.int32, (2 * _D, 2 * _D), 1)
    ident = (ii == jj).astype(jnp.float32)
    o_ref[...] = lax.dot_general(x, ident, (((0,), (0,)), ((), ())))


def _pack_split(table_t, half):
    n = table_t.shape[1]
    wblk = min(8192, half)
    nblk = half // wblk
    bmax = (n - 1) // wblk

    return pl.pallas_call(
        _transpose_body,
        grid=(nblk,),
        in_specs=[
            pl.BlockSpec((_D, wblk), lambda i: (0, i)),
            pl.BlockSpec((_D, wblk), lambda i: (0, jnp.minimum(i + nblk, bmax))),
        ],
        out_specs=pl.BlockSpec((wblk, 2 * _D), lambda i: (i, 0)),
        out_shape=jax.ShapeDtypeStruct((half, 2 * _D), jnp.float32),
    )(table_t, table_t)


def _sc_body(hq, tq, rq, hp, tp, rp, ent_hbm, rel_hbm, out_hbm,
             qh_v, qt_v, qr_v, ph_v, pt_v, pr_v,
             rows_h, rows_t, rows_r, p_v, sem):
    wid = lax.axis_index("s") * _NC + lax.axis_index("c")

    pltpu.sync_copy(hq.at[wid], qh_v)
    pltpu.sync_copy(tq.at[wid], qt_v)
    pltpu.sync_copy(rq.at[wid], qr_v)
    pltpu.sync_copy(hp.at[wid], ph_v)
    pltpu.sync_copy(tp.at[wid], pt_v)
    pltpu.sync_copy(rp.at[wid], pr_v)

    for c in range(_NCHUNK):
        cps = [
            pltpu.async_copy(ent_hbm.at[qh_v.at[c]], rows_h, sem),
            pltpu.async_copy(ent_hbm.at[qt_v.at[c]], rows_t, sem),
            pltpu.async_copy(rel_hbm.at[qr_v.at[c]], rows_r, sem),
        ]
        for cp in cps:
            cp.wait()

        def group(g, _):
            gsl = pl.ds(g * _L, _L)
            pvh = ph_v[c, gsl]
            pvt = pt_v[c, gsl]
            pvr = pr_v[c, gsl]
            for rr in range(_L):
                row = g * _L + rr
                oh = pvh[rr]
                ot = pvt[rr]
                orr = pvr[rr]
                s = None
                for k in range(_D // _L):
                    dh = rows_h[row, pl.ds(oh + k * _L, _L)]
                    dr = rows_r[row, pl.ds(orr + k * _L, _L)]
                    dt = rows_t[row, pl.ds(ot + k * _L, _L)]
                    d = dh + dr - dt
                    sq = d * d
                    s = sq if s is None else s + sq
                p_v[pl.ds((c * _CHUNK + row) * _L, _L)] = s
            return 0

        lax.fori_loop(0, _GPC, group, 0)

    pltpu.sync_copy(p_v, out_hbm.at[pl.ds(wid * _RPW * _L, _RPW * _L)])


@functools.partial(
    pl.kernel,
    out_type=jax.ShapeDtypeStruct((_B * _L,), jnp.float32),
    mesh=plsc.VectorSubcoreMesh(core_axis_name="c", subcore_axis_name="s"),
    scratch_types=[
        pltpu.VMEM((_NCHUNK, _CHUNK), jnp.int32),
        pltpu.VMEM((_NCHUNK, _CHUNK), jnp.int32),
        pltpu.VMEM((_NCHUNK, _CHUNK), jnp.int32),
        pltpu.VMEM((_NCHUNK, _CHUNK), jnp.int32),
        pltpu.VMEM((_NCHUNK, _CHUNK), jnp.int32),
        pltpu.VMEM((_NCHUNK, _CHUNK), jnp.int32),
        pltpu.VMEM((_CHUNK, 2 * _D), jnp.float32),
        pltpu.VMEM((_CHUNK, 2 * _D), jnp.float32),
        pltpu.VMEM((_CHUNK, 2 * _D), jnp.float32),
        pltpu.VMEM((_RPW * _L,), jnp.float32),
        pltpu.SemaphoreType.DMA,
    ],
)
def _transe_partials(hq, tq, rq, hp, tp, rp, ent_hbm, rel_hbm, out_hbm,
                     qh_v, qt_v, qr_v, ph_v, pt_v, pr_v,
                     rows_h, rows_t, rows_r, p_v, sem):
    _sc_body(hq, tq, rq, hp, tp, rp, ent_hbm, rel_hbm, out_hbm,
             qh_v, qt_v, qr_v, ph_v, pt_v, pr_v,
             rows_h, rows_t, rows_r, p_v, sem)


def _fold_body(p_ref, o_ref):
    o_ref[...] = jnp.sqrt(jnp.sum(p_ref[...], axis=-1))


_fold_sqrt = pl.pallas_call(
    _fold_body,
    out_shape=jax.ShapeDtypeStruct((_B,), jnp.float32),
)


def kernel(h, r, t, batch_size, ent_emb, rel_emb):
    del batch_size
    h = h.astype(jnp.int32)
    t = t.astype(jnp.int32)
    r = r.astype(jnp.int32)
    shape3 = (_NW, _NCHUNK, _CHUNK)
    ehalf = 1 << 19
    rhalf = 1 << 9
    hq = (h & (ehalf - 1)).reshape(shape3)
    tq = (t & (ehalf - 1)).reshape(shape3)
    rq = (r & (rhalf - 1)).reshape(shape3)
    hp = ((h >> 19) * _D).reshape(shape3)
    tp = ((t >> 19) * _D).reshape(shape3)
    rp = ((r >> 9) * _D).reshape(shape3)
    ent2 = _pack_split(ent_emb.T, ehalf)
    rel2 = _pack_split(rel_emb.T, rhalf)
    partials = _transe_partials(hq, tq, rq, hp, tp, rp, ent2, rel2)
    score = _fold_sqrt(partials.reshape(_B, _L))
    return score[:_HALF], score[_HALF:]


# speedup vs baseline: 3.6450x; 1.0222x over previous
import functools

import jax
import jax.numpy as jnp
from jax import lax
from jax.experimental import pallas as pl
from jax.experimental.pallas import tpu as pltpu
from jax.experimental.pallas import tpu_sc as plsc

_B = 16384
_D = 64
_L = 16
_HALF = 8192
_NC = 2
_NS = 16
_NW = _NC * _NS
_RPW = _B // _NW
_CHUNK = 128
_NCHUNK = _RPW // _CHUNK
_GPC = _CHUNK // _L

def _transpose_body(xa_ref, xb_ref, o_ref):
    x = jnp.concatenate([xa_ref[...], xb_ref[...]], axis=0)
    ii = lax.broadcasted_iota(jnp.int32, (2 * _D, 2 * _D), 0)
    jj = lax.broadcasted_iota(jnp.int32, (2 * _D, 2 * _D), 1)
    ident = (ii == jj).astype(jnp.float32)
    o_ref[...] = lax.dot_general(x, ident, (((0,), (0,)), ((), ())))


def _pack_split(table_t, half):
    n = table_t.shape[1]
    wblk = min(16384, half)
    nblk = half // wblk
    bmax = (n - 1) // wblk

    return pl.pallas_call(
        _transpose_body,
        grid=(nblk,),
        in_specs=[
            pl.BlockSpec((_D, wblk), lambda i: (0, i)),
            pl.BlockSpec((_D, wblk), lambda i: (0, jnp.minimum(i + nblk, bmax))),
        ],
        out_specs=pl.BlockSpec((wblk, 2 * _D), lambda i: (i, 0)),
        out_shape=jax.ShapeDtypeStruct((half, 2 * _D), jnp.float32),
    )(table_t, table_t)


def _sc_body(hq, tq, rq, hp, tp, rp, ent_hbm, rel_hbm, out_hbm,
             qh_v, qt_v, qr_v, ph_v, pt_v, pr_v,
             rows_h, rows_t, rows_r, p_v, sem):
    wid = lax.axis_index("s") * _NC + lax.axis_index("c")

    pltpu.sync_copy(hq.at[wid], qh_v)
    pltpu.sync_copy(tq.at[wid], qt_v)
    pltpu.sync_copy(rq.at[wid], qr_v)
    pltpu.sync_copy(hp.at[wid], ph_v)
    pltpu.sync_copy(tp.at[wid], pt_v)
    pltpu.sync_copy(rp.at[wid], pr_v)

    for c in range(_NCHUNK):
        cps = [
            pltpu.async_copy(ent_hbm.at[qh_v.at[c]], rows_h, sem),
            pltpu.async_copy(ent_hbm.at[qt_v.at[c]], rows_t, sem),
            pltpu.async_copy(rel_hbm.at[qr_v.at[c]], rows_r, sem),
        ]
        for cp in cps:
            cp.wait()

        def group(g, _):
            gsl = pl.ds(g * _L, _L)
            pvh = ph_v[c, gsl]
            pvt = pt_v[c, gsl]
            pvr = pr_v[c, gsl]
            for rr in range(_L):
                row = g * _L + rr
                oh = pvh[rr]
                ot = pvt[rr]
                orr = pvr[rr]
                s = None
                for k in range(_D // _L):
                    dh = rows_h[row, pl.ds(oh + k * _L, _L)]
                    dr = rows_r[row, pl.ds(orr + k * _L, _L)]
                    dt = rows_t[row, pl.ds(ot + k * _L, _L)]
                    d = dh + dr - dt
                    sq = d * d
                    s = sq if s is None else s + sq
                p_v[pl.ds((c * _CHUNK + row) * _L, _L)] = s
            return 0

        lax.fori_loop(0, _GPC, group, 0)

    pltpu.sync_copy(p_v, out_hbm.at[pl.ds(wid * _RPW * _L, _RPW * _L)])


@functools.partial(
    pl.kernel,
    out_type=jax.ShapeDtypeStruct((_B * _L,), jnp.float32),
    mesh=plsc.VectorSubcoreMesh(core_axis_name="c", subcore_axis_name="s"),
    scratch_types=[
        pltpu.VMEM((_NCHUNK, _CHUNK), jnp.int32),
        pltpu.VMEM((_NCHUNK, _CHUNK), jnp.int32),
        pltpu.VMEM((_NCHUNK, _CHUNK), jnp.int32),
        pltpu.VMEM((_NCHUNK, _CHUNK), jnp.int32),
        pltpu.VMEM((_NCHUNK, _CHUNK), jnp.int32),
        pltpu.VMEM((_NCHUNK, _CHUNK), jnp.int32),
        pltpu.VMEM((_CHUNK, 2 * _D), jnp.float32),
        pltpu.VMEM((_CHUNK, 2 * _D), jnp.float32),
        pltpu.VMEM((_CHUNK, 2 * _D), jnp.float32),
        pltpu.VMEM((_RPW * _L,), jnp.float32),
        pltpu.SemaphoreType.DMA,
    ],
)
def _transe_partials(hq, tq, rq, hp, tp, rp, ent_hbm, rel_hbm, out_hbm,
                     qh_v, qt_v, qr_v, ph_v, pt_v, pr_v,
                     rows_h, rows_t, rows_r, p_v, sem):
    _sc_body(hq, tq, rq, hp, tp, rp, ent_hbm, rel_hbm, out_hbm,
             qh_v, qt_v, qr_v, ph_v, pt_v, pr_v,
             rows_h, rows_t, rows_r, p_v, sem)


def _fold_body(p_ref, o_ref):
    o_ref[...] = jnp.sqrt(jnp.sum(p_ref[...], axis=-1))


_fold_sqrt = pl.pallas_call(
    _fold_body,
    out_shape=jax.ShapeDtypeStruct((_B,), jnp.float32),
)


def kernel(h, r, t, batch_size, ent_emb, rel_emb):
    del batch_size
    h = h.astype(jnp.int32)
    t = t.astype(jnp.int32)
    r = r.astype(jnp.int32)
    shape3 = (_NW, _NCHUNK, _CHUNK)
    ehalf = 1 << 19
    rhalf = 1 << 9
    hq = (h & (ehalf - 1)).reshape(shape3)
    tq = (t & (ehalf - 1)).reshape(shape3)
    rq = (r & (rhalf - 1)).reshape(shape3)
    hp = ((h >> 19) * _D).reshape(shape3)
    tp = ((t >> 19) * _D).reshape(shape3)
    rp = ((r >> 9) * _D).reshape(shape3)
    ent2 = _pack_split(ent_emb.T, ehalf)
    rel2 = _pack_split(rel_emb.T, rhalf)
    partials = _transe_partials(hq, tq, rq, hp, tp, rp, ent2, rel2)
    score = _fold_sqrt(partials.reshape(_B, _L))
    return score[:_HALF], score[_HALF:]


# speedup vs baseline: 4.3037x; 1.1807x over previous
import functools

import jax
import jax.numpy as jnp
from jax import lax
from jax.experimental import pallas as pl
from jax.experimental.pallas import tpu as pltpu
from jax.experimental.pallas import tpu_sc as plsc

_B = 16384
_D = 64
_L = 16
_HALF = 8192
_NC = 2
_NS = 16
_NW = _NC * _NS
_RPW = _B // _NW
_CHUNK = 128
_NCHUNK = _RPW // _CHUNK
_GPC = _CHUNK // _L

def _transpose_body(xa_ref, xb_ref, xc_ref, xd_ref, o_ref):
    x = jnp.concatenate(
        [xa_ref[...], xb_ref[...], xc_ref[...], xd_ref[...]], axis=0)
    kk = lax.broadcasted_iota(jnp.int32, (4 * _D, 2 * _D), 0)
    ll = lax.broadcasted_iota(jnp.int32, (4 * _D, 2 * _D), 1)
    se = (kk == 2 * ll).astype(jnp.float32)
    so = (kk == 2 * ll + 1).astype(jnp.float32)
    ye = lax.dot_general(x, se, (((0,), (0,)), ((), ())))
    yo = lax.dot_general(x, so, (((0,), (0,)), ((), ())))
    eu = lax.bitcast_convert_type(ye.astype(jnp.bfloat16), jnp.uint16)
    ou = lax.bitcast_convert_type(yo.astype(jnp.bfloat16), jnp.uint16)
    packed = eu.astype(jnp.uint32) | (ou.astype(jnp.uint32) << 16)
    o_ref[...] = lax.bitcast_convert_type(packed, jnp.int32)


def _pack_split(table_t, quad):
    n = table_t.shape[1]
    wblk = min(16384, quad)
    nblk = quad // wblk
    bmax = (n - 1) // wblk

    def mk_map(q):
        return lambda i: (0, jnp.minimum(i + q * nblk, bmax))

    return pl.pallas_call(
        _transpose_body,
        grid=(nblk,),
        in_specs=[pl.BlockSpec((_D, wblk), mk_map(q)) for q in range(4)],
        out_specs=pl.BlockSpec((wblk, 2 * _D), lambda i: (i, 0)),
        out_shape=jax.ShapeDtypeStruct((quad, 2 * _D), jnp.int32),
    )(table_t, table_t, table_t, table_t)


def _sc_body(hq, tq, rq, hp, tp, rp, ent_hbm, rel_hbm, out_hbm,
             qh_v, qt_v, qr_v, ph_v, pt_v, pr_v,
             rows_h, rows_t, rows_r, p_v, sem):
    wid = lax.axis_index("s") * _NC + lax.axis_index("c")

    pltpu.sync_copy(hq.at[wid], qh_v)
    pltpu.sync_copy(tq.at[wid], qt_v)
    pltpu.sync_copy(rq.at[wid], qr_v)
    pltpu.sync_copy(hp.at[wid], ph_v)
    pltpu.sync_copy(tp.at[wid], pt_v)
    pltpu.sync_copy(rp.at[wid], pr_v)

    for c in range(_NCHUNK):
        cps = [
            pltpu.async_copy(ent_hbm.at[qh_v.at[c]], rows_h, sem),
            pltpu.async_copy(ent_hbm.at[qt_v.at[c]], rows_t, sem),
            pltpu.async_copy(rel_hbm.at[qr_v.at[c]], rows_r, sem),
        ]
        for cp in cps:
            cp.wait()

        def group(g, _):
            gsl = pl.ds(g * _L, _L)
            pvh = ph_v[c, gsl]
            pvt = pt_v[c, gsl]
            pvr = pr_v[c, gsl]
            for rr in range(_L):
                row = g * _L + rr
                oh = pl.multiple_of(pvh[rr], 32)
                ot = pl.multiple_of(pvt[rr], 32)
                orr = pl.multiple_of(pvr[rr], 32)
                s = None
                for k in range(2):
                    bh = rows_h[row, pl.ds(oh + k * _L, _L)]
                    br = rows_r[row, pl.ds(orr + k * _L, _L)]
                    bt = rows_t[row, pl.ds(ot + k * _L, _L)]
                    he = lax.bitcast_convert_type(lax.shift_left(bh, 16), jnp.float32)
                    re = lax.bitcast_convert_type(lax.shift_left(br, 16), jnp.float32)
                    te = lax.bitcast_convert_type(lax.shift_left(bt, 16), jnp.float32)
                    m = jnp.int32(-65536)
                    ho = lax.bitcast_convert_type(bh & m, jnp.float32)
                    ro = lax.bitcast_convert_type(br & m, jnp.float32)
                    to = lax.bitcast_convert_type(bt & m, jnp.float32)
                    de = he + re - te
                    do = ho + ro - to
                    sq = de * de + do * do
                    s = sq if s is None else s + sq
                p_v[pl.ds((c * _CHUNK + row) * _L, _L)] = s
            return 0

        lax.fori_loop(0, _GPC, group, 0)

    pltpu.sync_copy(p_v, out_hbm.at[pl.ds(wid * _RPW * _L, _RPW * _L)])


@functools.partial(
    pl.kernel,
    out_type=jax.ShapeDtypeStruct((_B * _L,), jnp.float32),
    mesh=plsc.VectorSubcoreMesh(core_axis_name="c", subcore_axis_name="s"),
    scratch_types=[
        pltpu.VMEM((_NCHUNK, _CHUNK), jnp.int32),
        pltpu.VMEM((_NCHUNK, _CHUNK), jnp.int32),
        pltpu.VMEM((_NCHUNK, _CHUNK), jnp.int32),
        pltpu.VMEM((_NCHUNK, _CHUNK), jnp.int32),
        pltpu.VMEM((_NCHUNK, _CHUNK), jnp.int32),
        pltpu.VMEM((_NCHUNK, _CHUNK), jnp.int32),
        pltpu.VMEM((_CHUNK, 2 * _D), jnp.int32),
        pltpu.VMEM((_CHUNK, 2 * _D), jnp.int32),
        pltpu.VMEM((_CHUNK, 2 * _D), jnp.int32),
        pltpu.VMEM((_RPW * _L,), jnp.float32),
        pltpu.SemaphoreType.DMA,
    ],
)
def _transe_partials(hq, tq, rq, hp, tp, rp, ent_hbm, rel_hbm, out_hbm,
                     qh_v, qt_v, qr_v, ph_v, pt_v, pr_v,
                     rows_h, rows_t, rows_r, p_v, sem):
    _sc_body(hq, tq, rq, hp, tp, rp, ent_hbm, rel_hbm, out_hbm,
             qh_v, qt_v, qr_v, ph_v, pt_v, pr_v,
             rows_h, rows_t, rows_r, p_v, sem)


def _fold_body(p_ref, o_ref):
    o_ref[...] = jnp.sqrt(jnp.sum(p_ref[...], axis=-1))


_fold_sqrt = pl.pallas_call(
    _fold_body,
    out_shape=jax.ShapeDtypeStruct((_B,), jnp.float32),
)


def kernel(h, r, t, batch_size, ent_emb, rel_emb):
    del batch_size
    h = h.astype(jnp.int32)
    t = t.astype(jnp.int32)
    r = r.astype(jnp.int32)
    shape3 = (_NW, _NCHUNK, _CHUNK)
    equad = 1 << 18
    rquad = 1 << 8
    hq = (h & (equad - 1)).reshape(shape3)
    tq = (t & (equad - 1)).reshape(shape3)
    rq = (r & (rquad - 1)).reshape(shape3)
    hp = ((h >> 18) * 32).reshape(shape3)
    tp = ((t >> 18) * 32).reshape(shape3)
    rp = ((r >> 8) * 32).reshape(shape3)
    ent2 = _pack_split(ent_emb.T, equad)
    rel2 = _pack_split(rel_emb.T, rquad)
    partials = _transe_partials(hq, tq, rq, hp, tp, rp, ent2, rel2)
    score = _fold_sqrt(partials.reshape(_B, _L))
    return score[:_HALF], score[_HALF:]


# speedup vs baseline: 4.5980x; 1.0684x over previous
import functools

import jax
import jax.numpy as jnp
from jax import lax
from jax.experimental import pallas as pl
from jax.experimental.pallas import tpu as pltpu
from jax.experimental.pallas import tpu_sc as plsc

_B = 16384
_D = 64
_L = 16
_HALF = 8192
_NC = 2
_NS = 16
_NW = _NC * _NS
_RPW = _B // _NW
_CHUNK = 128
_NCHUNK = _RPW // _CHUNK
_GPC = _CHUNK // _L

def _transpose_body(xa_ref, xb_ref, xc_ref, xd_ref, o_ref):
    x = jnp.concatenate(
        [xa_ref[...], xb_ref[...], xc_ref[...], xd_ref[...]], axis=0)
    kk = lax.broadcasted_iota(jnp.int32, (4 * _D, 2 * _D), 0)
    ll = lax.broadcasted_iota(jnp.int32, (4 * _D, 2 * _D), 1)
    se = (kk == 2 * ll).astype(jnp.float32)
    so = (kk == 2 * ll + 1).astype(jnp.float32)
    ye = lax.dot_general(x, se, (((0,), (0,)), ((), ())))
    yo = lax.dot_general(x, so, (((0,), (0,)), ((), ())))
    eu = lax.bitcast_convert_type(ye.astype(jnp.bfloat16), jnp.uint16)
    ou = lax.bitcast_convert_type(yo.astype(jnp.bfloat16), jnp.uint16)
    packed = eu.astype(jnp.uint32) | (ou.astype(jnp.uint32) << 16)
    o_ref[...] = lax.bitcast_convert_type(packed, jnp.int32)


def _pack_split(table_t, quad):
    n = table_t.shape[1]
    wblk = min(16384, quad)
    nblk = quad // wblk
    bmax = (n - 1) // wblk

    def mk_map(q):
        return lambda i: (0, jnp.minimum(i + q * nblk, bmax))

    return pl.pallas_call(
        _transpose_body,
        grid=(nblk,),
        in_specs=[pl.BlockSpec((_D, wblk), mk_map(q)) for q in range(4)],
        out_specs=pl.BlockSpec((wblk, 2 * _D), lambda i: (i, 0)),
        out_shape=jax.ShapeDtypeStruct((quad, 2 * _D), jnp.int32),
    )(table_t, table_t, table_t, table_t)


def _sc_body(hq, tq, rq, hp, tp, rp, ent_hbm, rel_hbm, out_hbm,
             qh_v, qt_v, qr_v, ph_v, pt_v, pr_v,
             rows_h, rows_t, rows_r, p_v, sem):
    wid = lax.axis_index("s") * _NC + lax.axis_index("c")

    pltpu.sync_copy(hq.at[wid], qh_v)
    pltpu.sync_copy(tq.at[wid], qt_v)
    pltpu.sync_copy(rq.at[wid], qr_v)
    pltpu.sync_copy(hp.at[wid], ph_v)
    pltpu.sync_copy(tp.at[wid], pt_v)
    pltpu.sync_copy(rp.at[wid], pr_v)

    for c in range(_NCHUNK):
        cps = [
            pltpu.async_copy(ent_hbm.at[qh_v.at[c]], rows_h, sem),
            pltpu.async_copy(ent_hbm.at[qt_v.at[c]], rows_t, sem),
            pltpu.async_copy(rel_hbm.at[qr_v.at[c]], rows_r, sem),
        ]
        for cp in cps:
            cp.wait()

        def group(g, _):
            gsl = pl.ds(g * _L, _L)
            pvh = ph_v[c, gsl]
            pvt = pt_v[c, gsl]
            pvr = pr_v[c, gsl]
            for rr in range(_L):
                row = g * _L + rr
                oh = pl.multiple_of(pvh[rr], 32)
                ot = pl.multiple_of(pvt[rr], 32)
                orr = pl.multiple_of(pvr[rr], 32)
                s = None
                for k in range(2):
                    bh = rows_h[row, pl.ds(oh + k * _L, _L)]
                    br = rows_r[row, pl.ds(orr + k * _L, _L)]
                    bt = rows_t[row, pl.ds(ot + k * _L, _L)]
                    he = lax.bitcast_convert_type(lax.shift_left(bh, 16), jnp.float32)
                    re = lax.bitcast_convert_type(lax.shift_left(br, 16), jnp.float32)
                    te = lax.bitcast_convert_type(lax.shift_left(bt, 16), jnp.float32)
                    m = jnp.int32(-65536)
                    ho = lax.bitcast_convert_type(bh & m, jnp.float32)
                    ro = lax.bitcast_convert_type(br & m, jnp.float32)
                    to = lax.bitcast_convert_type(bt & m, jnp.float32)
                    de = he + re - te
                    do = ho + ro - to
                    sq = de * de + do * do
                    s = sq if s is None else s + sq
                idx = c * _CHUNK + row
                p_v[idx >> 3, pl.ds(pl.multiple_of((idx & 7) * _L, _L), _L)] = s
            return 0

        lax.fori_loop(0, _GPC, group, 0)

    pltpu.sync_copy(p_v, out_hbm.at[pl.ds(wid * (_RPW * _L // 128), _RPW * _L // 128)])


@functools.partial(
    pl.kernel,
    out_type=jax.ShapeDtypeStruct((_B * _L // 128, 128), jnp.float32),
    mesh=plsc.VectorSubcoreMesh(core_axis_name="c", subcore_axis_name="s"),
    scratch_types=[
        pltpu.VMEM((_NCHUNK, _CHUNK), jnp.int32),
        pltpu.VMEM((_NCHUNK, _CHUNK), jnp.int32),
        pltpu.VMEM((_NCHUNK, _CHUNK), jnp.int32),
        pltpu.VMEM((_NCHUNK, _CHUNK), jnp.int32),
        pltpu.VMEM((_NCHUNK, _CHUNK), jnp.int32),
        pltpu.VMEM((_NCHUNK, _CHUNK), jnp.int32),
        pltpu.VMEM((_CHUNK, 2 * _D), jnp.int32),
        pltpu.VMEM((_CHUNK, 2 * _D), jnp.int32),
        pltpu.VMEM((_CHUNK, 2 * _D), jnp.int32),
        pltpu.VMEM((_RPW * _L // 128, 128), jnp.float32),
        pltpu.SemaphoreType.DMA,
    ],
)
def _transe_partials(hq, tq, rq, hp, tp, rp, ent_hbm, rel_hbm, out_hbm,
                     qh_v, qt_v, qr_v, ph_v, pt_v, pr_v,
                     rows_h, rows_t, rows_r, p_v, sem):
    _sc_body(hq, tq, rq, hp, tp, rp, ent_hbm, rel_hbm, out_hbm,
             qh_v, qt_v, qr_v, ph_v, pt_v, pr_v,
             rows_h, rows_t, rows_r, p_v, sem)


def _fold_body(p_ref, o_ref):
    cc = lax.broadcasted_iota(jnp.int32, (128, 8), 0)
    mm = lax.broadcasted_iota(jnp.int32, (128, 8), 1)
    g = ((cc >> 4) == mm).astype(jnp.float32)
    o_ref[...] = jnp.sqrt(jnp.dot(p_ref[...], g))


_fold_sqrt = pl.pallas_call(
    _fold_body,
    out_shape=jax.ShapeDtypeStruct((_B * _L // 128, 8), jnp.float32),
)


def kernel(h, r, t, batch_size, ent_emb, rel_emb):
    del batch_size
    h = h.astype(jnp.int32)
    t = t.astype(jnp.int32)
    r = r.astype(jnp.int32)
    shape3 = (_NW, _NCHUNK, _CHUNK)
    equad = 1 << 18
    rquad = 1 << 8
    hq = (h & (equad - 1)).reshape(shape3)
    tq = (t & (equad - 1)).reshape(shape3)
    rq = (r & (rquad - 1)).reshape(shape3)
    hp = ((h >> 18) * 32).reshape(shape3)
    tp = ((t >> 18) * 32).reshape(shape3)
    rp = ((r >> 8) * 32).reshape(shape3)
    ent2 = _pack_split(ent_emb.T, equad)
    rel2 = _pack_split(rel_emb.T, rquad)
    partials = _transe_partials(hq, tq, rq, hp, tp, rp, ent2, rel2)
    score = _fold_sqrt(partials).reshape(_B)
    return score[:_HALF], score[_HALF:]


# speedup vs baseline: 4.6900x; 1.0200x over previous
import functools

import jax
import jax.numpy as jnp
from jax import lax
from jax.experimental import pallas as pl
from jax.experimental.pallas import tpu as pltpu
from jax.experimental.pallas import tpu_sc as plsc

_B = 16384
_D = 64
_L = 16
_HALF = 8192
_NC = 2
_NS = 16
_NW = _NC * _NS
_RPW = _B // _NW
_CHUNK = 128
_NCHUNK = _RPW // _CHUNK
_GPC = _CHUNK // _L

def _transpose_body(xa_ref, xb_ref, xc_ref, xd_ref, o_ref):
    x = jnp.concatenate(
        [xa_ref[...], xb_ref[...], xc_ref[...], xd_ref[...]], axis=0)
    kk = lax.broadcasted_iota(jnp.int32, (4 * _D, 2 * _D), 0)
    ll = lax.broadcasted_iota(jnp.int32, (4 * _D, 2 * _D), 1)
    se = (kk == 2 * ll).astype(jnp.float32)
    so = (kk == 2 * ll + 1).astype(jnp.float32)
    ye = lax.dot_general(x, se, (((0,), (0,)), ((), ())))
    yo = lax.dot_general(x, so, (((0,), (0,)), ((), ())))
    eu = lax.bitcast_convert_type(ye.astype(jnp.bfloat16), jnp.uint16)
    ou = lax.bitcast_convert_type(yo.astype(jnp.bfloat16), jnp.uint16)
    packed = eu.astype(jnp.uint32) | (ou.astype(jnp.uint32) << 16)
    o_ref[...] = lax.bitcast_convert_type(packed, jnp.int32)


def _pack_split(table_t, quad):
    n = table_t.shape[1]
    wblk = min(16384, quad)
    nblk = quad // wblk
    bmax = (n - 1) // wblk

    def mk_map(q):
        return lambda i: (0, jnp.minimum(i + q * nblk, bmax))

    return pl.pallas_call(
        _transpose_body,
        grid=(nblk,),
        in_specs=[pl.BlockSpec((_D, wblk), mk_map(q)) for q in range(4)],
        out_specs=pl.BlockSpec((wblk, 2 * _D), lambda i: (i, 0)),
        out_shape=jax.ShapeDtypeStruct((quad, 2 * _D), jnp.int32),
    )(table_t, table_t, table_t, table_t)


def _sc_body(hq, tq, rq, hp, tp, rp, ent_hbm, rel_hbm, out_hbm,
             qh_v, qt_v, qr_v, ph_v, pt_v, pr_v,
             rows_h0, rows_t0, rows_r0, rows_h1, rows_t1, rows_r1,
             p_v, sem0, sem1):
    wid = lax.axis_index("s") * _NC + lax.axis_index("c")

    pltpu.sync_copy(hq.at[wid], qh_v)
    pltpu.sync_copy(tq.at[wid], qt_v)
    pltpu.sync_copy(rq.at[wid], qr_v)
    pltpu.sync_copy(hp.at[wid], ph_v)
    pltpu.sync_copy(tp.at[wid], pt_v)
    pltpu.sync_copy(rp.at[wid], pr_v)

    rows_h = (rows_h0, rows_h1)
    rows_t = (rows_t0, rows_t1)
    rows_r = (rows_r0, rows_r1)
    sem = (sem0, sem1)

    def fire(c):
        b = c & 1
        return [
            pltpu.async_copy(ent_hbm.at[qh_v.at[c]], rows_h[b], sem[b]),
            pltpu.async_copy(ent_hbm.at[qt_v.at[c]], rows_t[b], sem[b]),
            pltpu.async_copy(rel_hbm.at[qr_v.at[c]], rows_r[b], sem[b]),
        ]

    pend = {0: fire(0)}
    for c in range(_NCHUNK):
        if c + 1 < _NCHUNK:
            pend[c + 1] = fire(c + 1)
        for cp in pend.pop(c):
            cp.wait()
        rh, rt, rr_b = rows_h[c & 1], rows_t[c & 1], rows_r[c & 1]

        def group(g, _, rows_h=rh, rows_t=rt, rows_r=rr_b):
            gsl = pl.ds(g * _L, _L)
            pvh = ph_v[c, gsl]
            pvt = pt_v[c, gsl]
            pvr = pr_v[c, gsl]
            for rr in range(_L):
                row = g * _L + rr
                oh = pl.multiple_of(pvh[rr], 32)
                ot = pl.multiple_of(pvt[rr], 32)
                orr = pl.multiple_of(pvr[rr], 32)
                s = None
                for k in range(2):
                    bh = rows_h[row, pl.ds(oh + k * _L, _L)]
                    br = rows_r[row, pl.ds(orr + k * _L, _L)]
                    bt = rows_t[row, pl.ds(ot + k * _L, _L)]
                    he = lax.bitcast_convert_type(lax.shift_left(bh, 16), jnp.float32)
                    re = lax.bitcast_convert_type(lax.shift_left(br, 16), jnp.float32)
                    te = lax.bitcast_convert_type(lax.shift_left(bt, 16), jnp.float32)
                    m = jnp.int32(-65536)
                    ho = lax.bitcast_convert_type(bh & m, jnp.float32)
                    ro = lax.bitcast_convert_type(br & m, jnp.float32)
                    to = lax.bitcast_convert_type(bt & m, jnp.float32)
                    de = he + re - te
                    do = ho + ro - to
                    sq = de * de + do * do
                    s = sq if s is None else s + sq
                idx = c * _CHUNK + row
                p_v[idx >> 3, pl.ds(pl.multiple_of((idx & 7) * _L, _L), _L)] = s
            return 0

        lax.fori_loop(0, _GPC, group, 0)

    pltpu.sync_copy(p_v, out_hbm.at[pl.ds(wid * (_RPW * _L // 128), _RPW * _L // 128)])


@functools.partial(
    pl.kernel,
    out_type=jax.ShapeDtypeStruct((_B * _L // 128, 128), jnp.float32),
    mesh=plsc.VectorSubcoreMesh(core_axis_name="c", subcore_axis_name="s"),
    scratch_types=[
        pltpu.VMEM((_NCHUNK, _CHUNK), jnp.int32),
        pltpu.VMEM((_NCHUNK, _CHUNK), jnp.int32),
        pltpu.VMEM((_NCHUNK, _CHUNK), jnp.int32),
        pltpu.VMEM((_NCHUNK, _CHUNK), jnp.int32),
        pltpu.VMEM((_NCHUNK, _CHUNK), jnp.int32),
        pltpu.VMEM((_NCHUNK, _CHUNK), jnp.int32),
        pltpu.VMEM((_CHUNK, 2 * _D), jnp.int32),
        pltpu.VMEM((_CHUNK, 2 * _D), jnp.int32),
        pltpu.VMEM((_CHUNK, 2 * _D), jnp.int32),
        pltpu.VMEM((_CHUNK, 2 * _D), jnp.int32),
        pltpu.VMEM((_CHUNK, 2 * _D), jnp.int32),
        pltpu.VMEM((_CHUNK, 2 * _D), jnp.int32),
        pltpu.VMEM((_RPW * _L // 128, 128), jnp.float32),
        pltpu.SemaphoreType.DMA,
        pltpu.SemaphoreType.DMA,
    ],
)
def _transe_partials(hq, tq, rq, hp, tp, rp, ent_hbm, rel_hbm, out_hbm,
                     qh_v, qt_v, qr_v, ph_v, pt_v, pr_v,
                     rows_h0, rows_t0, rows_r0, rows_h1, rows_t1, rows_r1,
                     p_v, sem0, sem1):
    _sc_body(hq, tq, rq, hp, tp, rp, ent_hbm, rel_hbm, out_hbm,
             qh_v, qt_v, qr_v, ph_v, pt_v, pr_v,
             rows_h0, rows_t0, rows_r0, rows_h1, rows_t1, rows_r1,
             p_v, sem0, sem1)


def _fold_body(p_ref, o_ref):
    cc = lax.broadcasted_iota(jnp.int32, (128, 8), 0)
    mm = lax.broadcasted_iota(jnp.int32, (128, 8), 1)
    g = ((cc >> 4) == mm).astype(jnp.float32)
    o_ref[...] = jnp.sqrt(jnp.dot(p_ref[...], g))


_fold_sqrt = pl.pallas_call(
    _fold_body,
    out_shape=jax.ShapeDtypeStruct((_B * _L // 128, 8), jnp.float32),
)


def kernel(h, r, t, batch_size, ent_emb, rel_emb):
    del batch_size
    h = h.astype(jnp.int32)
    t = t.astype(jnp.int32)
    r = r.astype(jnp.int32)
    shape3 = (_NW, _NCHUNK, _CHUNK)
    equad = 1 << 18
    rquad = 1 << 8
    hq = (h & (equad - 1)).reshape(shape3)
    tq = (t & (equad - 1)).reshape(shape3)
    rq = (r & (rquad - 1)).reshape(shape3)
    hp = ((h >> 18) * 32).reshape(shape3)
    tp = ((t >> 18) * 32).reshape(shape3)
    rp = ((r >> 8) * 32).reshape(shape3)
    ent2 = _pack_split(ent_emb.T, equad)
    rel2 = _pack_split(rel_emb.T, rquad)
    partials = _transe_partials(hq, tq, rq, hp, tp, rp, ent2, rel2)
    score = _fold_sqrt(partials).reshape(_B)
    return score[:_HALF], score[_HALF:]
